# Initial kernel scaffold; baseline (speedup 1.0000x reference)
import jax
import jax.numpy as jnp
from jax.experimental import pallas as pl


def kernel(x, p, n2, W1, b1, gamma, beta, W2, b2):
    raise NotImplementedError("write your pallas kernel here")


# speedup vs baseline: 14.0203x; 14.0203x over previous
import functools

import jax
import jax.numpy as jnp
from jax import lax
from jax.experimental import pallas as pl
from jax.experimental.pallas import tpu as pltpu
from jax.experimental.pallas import tpu_sc as plsc

KNB = 16
DIN = 64
DOUT = 128
N2 = 2048
QT = 128
RT = 2048
MT = 128

_HIGH = lax.Precision.HIGHEST


def _topk_indices(p2, pT):
    B, _, N = pT.shape

    def body(p2_ref, pT_ref, idx_ref, d2_ref):
        b = pl.program_id(0)
        q = p2_ref[0]
        qx, qy, qz = q[:, 0:1], q[:, 1:2], q[:, 2:3]
        px = pT_ref[0, 0:1, :]
        py = pT_ref[0, 1:2, :]
        pz = pT_ref[0, 2:3, :]
        qn = qx * qx + qy * qy + qz * qz
        pn = px * px + py * py + pz * pz
        cross = lax.dot_general(q, pT_ref[0], (((1,), (0,)), ((), ())),
                                preferred_element_type=jnp.float32)
        d2_ref[...] = (qn + pn) - 2.0 * cross
        lane = lax.broadcasted_iota(jnp.int32, (QT, N), 1)
        off = b * N
        for k in range(KNB):
            d2 = d2_ref[...]
            m = jnp.min(d2, axis=1, keepdims=True)
            am = jnp.min(jnp.where(d2 <= m, lane, N), axis=1, keepdims=True)
            idx_ref[0, :, k:k + 1] = am + off
            d2_ref[...] = jnp.where(lane == am, jnp.inf, d2)

    return pl.pallas_call(
        body,
        grid=(B, N2 // QT),
        in_specs=[
            pl.BlockSpec((1, QT, 3), lambda b, qb: (b, qb, 0)),
            pl.BlockSpec((1, 3, N), lambda b, qb: (b, 0, 0)),
        ],
        out_specs=pl.BlockSpec((1, QT, KNB), lambda b, qb: (b, qb, 0)),
        out_shape=jax.ShapeDtypeStruct((B, N2, KNB), jnp.int32),
        scratch_shapes=[pltpu.VMEM((QT, N), jnp.float32)],
    )(p2, pT)


def _linear1(x_flat, W1, b1):
    V = x_flat.shape[0]

    def body(x_ref, w_ref, b_ref, out_ref):
        out_ref[...] = jnp.dot(
            x_ref[...], w_ref[...], precision=_HIGH,
            preferred_element_type=jnp.float32) + b_ref[...]

    return pl.pallas_call(
        body,
        grid=(V // RT,),
        in_specs=[
            pl.BlockSpec((RT, DIN), lambda i: (i, 0)),
            pl.BlockSpec((DIN, DOUT), lambda i: (0, 0)),
            pl.BlockSpec((1, DOUT), lambda i: (0, 0)),
        ],
        out_specs=pl.BlockSpec((RT, DOUT), lambda i: (i, 0)),
        out_shape=jax.ShapeDtypeStruct((V, DOUT), jnp.float32),
    )(x_flat, W1, b1.reshape(1, DOUT))


def _sc_gather(h_flat, idxf):
    info = plsc.get_sparse_core_info()
    nc, ns = info.num_cores, info.num_subcores
    nw = nc * ns
    M = idxf.shape[0]
    CH = 128
    rows_per_w = M // nw
    nch = rows_per_w // CH
    mesh = plsc.VectorSubcoreMesh(core_axis_name="c", subcore_axis_name="s")

    @functools.partial(
        pl.kernel,
        mesh=mesh,
        out_type=jax.ShapeDtypeStruct((M, DOUT), jnp.float32),
        scratch_types=[
            pltpu.VMEM((CH,), jnp.int32),
            pltpu.VMEM((CH, DOUT), jnp.float32),
            pltpu.SemaphoreType.DMA,
        ],
    )
    def gather_k(h_hbm, idx_hbm, out_hbm, idx_v, rows_v, sem):
        wid = lax.axis_index("s") * nc + lax.axis_index("c")
        base = wid * rows_per_w
        for c in range(nch):
            off = base + c * CH
            pltpu.sync_copy(idx_hbm.at[pl.ds(off, CH)], idx_v)
            pltpu.async_copy(h_hbm.at[idx_v], rows_v, sem).wait()
            pltpu.sync_copy(rows_v, out_hbm.at[pl.ds(off, CH)])

    return gather_k(h_flat, idxf)


def _stats(x2h):
    M = x2h.shape[0]

    def body(x_ref, s1_ref, s2_ref):
        @pl.when(pl.program_id(0) == 0)
        def _():
            s1_ref[...] = jnp.zeros_like(s1_ref)
            s2_ref[...] = jnp.zeros_like(s2_ref)

        xb = x_ref[...]
        s1_ref[...] += jnp.sum(xb, axis=0, keepdims=True)
        s2_ref[...] += jnp.sum(xb * xb, axis=0, keepdims=True)

    return pl.pallas_call(
        body,
        grid=(M // RT,),
        in_specs=[pl.BlockSpec((RT, DOUT), lambda i: (i, 0))],
        out_specs=[
            pl.BlockSpec((1, DOUT), lambda i: (0, 0)),
            pl.BlockSpec((1, DOUT), lambda i: (0, 0)),
        ],
        out_shape=[
            jax.ShapeDtypeStruct((1, DOUT), jnp.float32),
            jax.ShapeDtypeStruct((1, DOUT), jnp.float32),
        ],
    )(x2h)


def _mlp_maxpool(x2k, scale, shift, W2, b2p):
    BQ = x2k.shape[0]

    def body(x_ref, s_ref, t_ref, w2_ref, b2_ref, out_ref):
        s = s_ref[...]
        t = t_ref[...]
        w2 = w2_ref[...]
        acc = jnp.full((MT, DOUT), -jnp.inf, jnp.float32)
        for k in range(KNB):
            hk = jnp.maximum(x_ref[:, k, :] * s + t, 0.0)
            g = jnp.dot(hk, w2, precision=_HIGH,
                        preferred_element_type=jnp.float32)
            acc = jnp.maximum(acc, g)
        out_ref[...] = acc + b2_ref[...]

    return pl.pallas_call(
        body,
        grid=(BQ // MT,),
        in_specs=[
            pl.BlockSpec((MT, KNB, DOUT), lambda i: (i, 0, 0)),
            pl.BlockSpec((1, DOUT), lambda i: (0, 0)),
            pl.BlockSpec((1, DOUT), lambda i: (0, 0)),
            pl.BlockSpec((DOUT, DOUT), lambda i: (0, 0)),
            pl.BlockSpec((1, DOUT), lambda i: (0, 0)),
        ],
        out_specs=pl.BlockSpec((MT, DOUT), lambda i: (i, 0)),
        out_shape=jax.ShapeDtypeStruct((BQ, DOUT), jnp.float32),
    )(x2k, scale, shift, W2, b2p)


def kernel(x, p, n2, W1, b1, gamma, beta, W2, b2):
    B, N, _ = x.shape
    stride = N // N2
    p2 = p[:, ::stride, :]
    pT = jnp.transpose(p, (0, 2, 1))

    idx = _topk_indices(p2, pT)
    x_flat = x.reshape(B * N, DIN)
    idxf = idx.reshape(B * N2 * KNB)

    h1 = _linear1(x_flat, W1, b1)
    x2h = _sc_gather(h1, idxf)

    M = B * N2 * KNB
    s1, s2 = _stats(x2h)
    mean = s1[0] / M
    var = s2[0] / M - mean * mean
    scale = gamma / jnp.sqrt(var + 1e-5)
    shift = beta - mean * scale

    out = _mlp_maxpool(x2h.reshape(B * N2, KNB, DOUT),
                       scale.reshape(1, DOUT), shift.reshape(1, DOUT),
                       W2, b2.reshape(1, DOUT))
    return out.reshape(B, N2, DOUT), p2


# speedup vs baseline: 14.0275x; 1.0005x over previous
import functools

import jax
import jax.numpy as jnp
from jax import lax
from jax.experimental import pallas as pl
from jax.experimental.pallas import tpu as pltpu
from jax.experimental.pallas import tpu_sc as plsc

KNB = 16
DIN = 64
DOUT = 128
N2 = 2048
QT = 128
RT = 2048
MT = 128

_HIGH = lax.Precision.HIGHEST


def _topk_indices(p2, pT):
    B, _, N = pT.shape

    def body(p2_ref, pT_ref, idx_ref, d2_ref):
        b = pl.program_id(0)
        q = p2_ref[0]
        qx, qy, qz = q[:, 0:1], q[:, 1:2], q[:, 2:3]
        px = pT_ref[0, 0:1, :]
        py = pT_ref[0, 1:2, :]
        pz = pT_ref[0, 2:3, :]
        qn = qx * qx + qy * qy + qz * qz
        pn = px * px + py * py + pz * pz
        cross = lax.dot_general(q, pT_ref[0], (((1,), (0,)), ((), ())),
                                preferred_element_type=jnp.float32)
        d2_ref[...] = (qn + pn) - 2.0 * cross
        lane = lax.broadcasted_iota(jnp.int32, (1, N), 1)
        off = b * N
        for k in range(KNB):
            d2 = d2_ref[...]
            m = jnp.min(d2, axis=1, keepdims=True)
            am = jnp.min(jnp.where(d2 <= m, lane, N), axis=1, keepdims=True)
            idx_ref[0, :, k:k + 1] = am + off
            if k + 1 < KNB:
                d2_ref[...] = jnp.where(lane == am, jnp.inf, d2)

    return pl.pallas_call(
        body,
        grid=(B, N2 // QT),
        in_specs=[
            pl.BlockSpec((1, QT, 3), lambda b, qb: (b, qb, 0)),
            pl.BlockSpec((1, 3, N), lambda b, qb: (b, 0, 0)),
        ],
        out_specs=pl.BlockSpec((1, QT, KNB), lambda b, qb: (b, qb, 0)),
        out_shape=jax.ShapeDtypeStruct((B, N2, KNB), jnp.int32),
        scratch_shapes=[pltpu.VMEM((QT, N), jnp.float32)],
    )(p2, pT)


def _linear1(x_flat, W1, b1):
    V = x_flat.shape[0]

    def body(x_ref, w_ref, b_ref, out_ref):
        out_ref[...] = jnp.dot(
            x_ref[...], w_ref[...], precision=_HIGH,
            preferred_element_type=jnp.float32) + b_ref[...]

    return pl.pallas_call(
        body,
        grid=(V // RT,),
        in_specs=[
            pl.BlockSpec((RT, DIN), lambda i: (i, 0)),
            pl.BlockSpec((DIN, DOUT), lambda i: (0, 0)),
            pl.BlockSpec((1, DOUT), lambda i: (0, 0)),
        ],
        out_specs=pl.BlockSpec((RT, DOUT), lambda i: (i, 0)),
        out_shape=jax.ShapeDtypeStruct((V, DOUT), jnp.float32),
    )(x_flat, W1, b1.reshape(1, DOUT))


def _sc_gather(h_flat, idxf):
    info = plsc.get_sparse_core_info()
    nc, ns = info.num_cores, info.num_subcores
    nw = nc * ns
    M = idxf.shape[0]
    CH = 128
    rows_per_w = M // nw
    nch = rows_per_w // CH
    mesh = plsc.VectorSubcoreMesh(core_axis_name="c", subcore_axis_name="s")

    @functools.partial(
        pl.kernel,
        mesh=mesh,
        out_type=jax.ShapeDtypeStruct((M, DOUT), jnp.float32),
        scratch_types=[
            pltpu.VMEM((CH,), jnp.int32),
            pltpu.VMEM((CH, DOUT), jnp.float32),
            pltpu.SemaphoreType.DMA,
        ],
    )
    def gather_k(h_hbm, idx_hbm, out_hbm, idx_v, rows_v, sem):
        wid = lax.axis_index("s") * nc + lax.axis_index("c")
        base = wid * rows_per_w
        for c in range(nch):
            off = base + c * CH
            pltpu.sync_copy(idx_hbm.at[pl.ds(off, CH)], idx_v)
            pltpu.async_copy(h_hbm.at[idx_v], rows_v, sem).wait()
            pltpu.sync_copy(rows_v, out_hbm.at[pl.ds(off, CH)])

    return gather_k(h_flat, idxf)


def _stats(x2h):
    M = x2h.shape[0]

    def body(x_ref, s1_ref, s2_ref):
        @pl.when(pl.program_id(0) == 0)
        def _():
            s1_ref[...] = jnp.zeros_like(s1_ref)
            s2_ref[...] = jnp.zeros_like(s2_ref)

        xb = x_ref[...]
        s1_ref[...] += jnp.sum(xb, axis=0, keepdims=True)
        s2_ref[...] += jnp.sum(xb * xb, axis=0, keepdims=True)

    return pl.pallas_call(
        body,
        grid=(M // RT,),
        in_specs=[pl.BlockSpec((RT, DOUT), lambda i: (i, 0))],
        out_specs=[
            pl.BlockSpec((1, DOUT), lambda i: (0, 0)),
            pl.BlockSpec((1, DOUT), lambda i: (0, 0)),
        ],
        out_shape=[
            jax.ShapeDtypeStruct((1, DOUT), jnp.float32),
            jax.ShapeDtypeStruct((1, DOUT), jnp.float32),
        ],
    )(x2h)


def _mlp_maxpool(x2k, scale, shift, W2, b2p):
    BQ = x2k.shape[0]

    def body(x_ref, s_ref, t_ref, w2_ref, b2_ref, out_ref):
        s = s_ref[...]
        t = t_ref[...]
        w2 = w2_ref[...]
        acc = jnp.full((MT, DOUT), -jnp.inf, jnp.float32)
        for k in range(KNB):
            hk = jnp.maximum(x_ref[:, k, :] * s + t, 0.0)
            g = jnp.dot(hk, w2, precision=_HIGH,
                        preferred_element_type=jnp.float32)
            acc = jnp.maximum(acc, g)
        out_ref[...] = acc + b2_ref[...]

    return pl.pallas_call(
        body,
        grid=(BQ // MT,),
        in_specs=[
            pl.BlockSpec((MT, KNB, DOUT), lambda i: (i, 0, 0)),
            pl.BlockSpec((1, DOUT), lambda i: (0, 0)),
            pl.BlockSpec((1, DOUT), lambda i: (0, 0)),
            pl.BlockSpec((DOUT, DOUT), lambda i: (0, 0)),
            pl.BlockSpec((1, DOUT), lambda i: (0, 0)),
        ],
        out_specs=pl.BlockSpec((MT, DOUT), lambda i: (i, 0)),
        out_shape=jax.ShapeDtypeStruct((BQ, DOUT), jnp.float32),
    )(x2k, scale, shift, W2, b2p)


def kernel(x, p, n2, W1, b1, gamma, beta, W2, b2):
    B, N, _ = x.shape
    stride = N // N2
    p2 = p[:, ::stride, :]
    pT = jnp.transpose(p, (0, 2, 1))

    idx = _topk_indices(p2, pT)
    x_flat = x.reshape(B * N, DIN)
    idxf = idx.reshape(B * N2 * KNB)

    h1 = _linear1(x_flat, W1, b1)
    x2h = _sc_gather(h1, idxf)

    M = B * N2 * KNB
    s1, s2 = _stats(x2h)
    mean = s1[0] / M
    var = s2[0] / M - mean * mean
    scale = gamma / jnp.sqrt(var + 1e-5)
    shift = beta - mean * scale

    out = _mlp_maxpool(x2h.reshape(B * N2, KNB, DOUT),
                       scale.reshape(1, DOUT), shift.reshape(1, DOUT),
                       W2, b2.reshape(1, DOUT))
    return out.reshape(B, N2, DOUT), p2
